# baseline (speedup 1.0000x reference)
import functools

import jax
import jax.numpy as jnp
from jax import lax
from jax.experimental import pallas as pl
from jax.experimental.pallas import tpu as pltpu
from jax.experimental.pallas import tpu_sc as plsc

CHUNK = 2048
MAXP = 7
LANES = 16
NUM_CHUNKS = 36
H = 3
C0 = NUM_CHUNKS - 2 * H
CPB = 2
TROWS = NUM_CHUNKS + 8
ZROW = TROWS - 1
ROWS_PER_TILE = CHUNK // 16
PIECE = 32
NPIECE = ROWS_PER_TILE // PIECE
EMB = 512
NG = EMB // LANES


def _tree_sum(vals):
    while len(vals) > 1:
        pairs = [vals[i] + vals[i + 1] for i in range(0, len(vals) - 1, 2)]
        if len(vals) % 2:
            pairs.append(vals[-1])
        vals = pairs
    return vals[0]


def _acc_piece(buf, acc_ref):
    for g in range(NG):
        vals = [buf[r, pl.ds(g * LANES, LANES)] for r in range(PIECE)]
        plsc.addupdate(acc_ref.at[pl.ds(g * LANES, LANES)], _tree_sum(vals))


def _zero_fill(ref, nvec):
    for i in range(nvec):
        ref[pl.ds(i * LANES, LANES)] = jnp.zeros((LANES,), jnp.float32)


def _sc_reduce_body(
    x_hbm, scale_hbm, out_hbm,
    buf0, buf1, buf2, buf3, acc_ref, comb, rowbuf, zbuf, scale_v, partials,
    sem0, sem1, sem2, sem3,
):
    core = lax.axis_index("c")
    tile = lax.axis_index("s")
    pltpu.sync_copy(scale_hbm, scale_v)
    bufs = (buf0, buf1, buf2, buf3)
    sems = (sem0, sem1, sem2, sem3)
    base0 = (C0 + core * H) * CHUNK + tile * ROWS_PER_TILE

    def src(chunk_j, p):
        return x_hbm.at[pl.ds(base0 + chunk_j * CHUNK + p * PIECE, PIECE)]

    for p in range(3):
        pltpu.async_copy(src(0, p), bufs[p], sems[p])

    def chunk_body(j, carry):
        pltpu.async_copy(src(j, 3), bufs[3], sems[3])
        _zero_fill(acc_ref, NG)
        for p in range(4):
            pltpu.make_async_copy(src(j, p), bufs[p], sems[p]).wait()
            _acc_piece(bufs[p], acc_ref)
            if p < 3:

                @pl.when(j < H - 1)
                def _(p=p):
                    pltpu.async_copy(src(j + 1, p), bufs[p], sems[p])
        pltpu.sync_copy(acc_ref, partials.at[j, tile])
        return carry

    lax.fori_loop(0, H, chunk_body, 0)
    plsc.subcore_barrier()

    @pl.when(tile < H)
    def _():
        pltpu.sync_copy(partials.at[tile], comb)
        sv = scale_v[...]
        for g in range(NG):
            vals = [comb[r, pl.ds(g * LANES, LANES)] for r in range(16)]
            rowbuf[pl.ds(g * LANES, LANES)] = _tree_sum(vals) * sv
        row = C0 + core * H + tile
        pltpu.sync_copy(rowbuf, out_hbm.at[pl.ds(row * EMB, EMB)])

    @pl.when((core == 0) & (tile == 15))
    def _():
        _zero_fill(zbuf, C0 * EMB // LANES)
        pltpu.sync_copy(zbuf, out_hbm.at[pl.ds(0, C0 * EMB)])

    @pl.when((core == 1) & (tile == 15))
    def _():
        npad = TROWS - NUM_CHUNKS
        _zero_fill(zbuf, npad * EMB // LANES)
        pltpu.sync_copy(
            zbuf.at[pl.ds(0, npad * EMB)],
            out_hbm.at[pl.ds(NUM_CHUNKS * EMB, npad * EMB)],
        )


def _tc_reduce_body(scale_ref, x_ref, out_ref):
    c = pl.program_id(0)

    @pl.when(c == 0)
    def _():
        out_ref[pl.ds(C0, TROWS - C0), :] = jnp.zeros(
            (TROWS - C0, EMB), jnp.float32
        )

    blk = x_ref[...].reshape(CPB, CHUNK, EMB)
    sums = jnp.sum(blk, axis=1) * scale_ref[0]
    for k in range(CPB):
        out_ref[pl.ds(c * CPB + k, 1), :] = sums[k : k + 1]


def _sc_assemble_body(
    batch, n_hbm, m_hbm, ma_hbm, mb_hbm, out_hbm,
    n_v, m_v, dup, rows_a, rows_b, stage, sem_a, sem_b,
):
    core = lax.axis_index("c")
    tile = lax.axis_index("s")

    @pl.when((core == 0) & (tile < batch))
    def _():
        pltpu.sync_copy(n_hbm, n_v)
        pltpu.sync_copy(m_hbm, m_v)
        n = n_v[...]
        lane = lax.iota(jnp.int32, LANES)
        start = jnp.zeros((LANES,), jnp.int32)
        for k in range(batch):
            bk = plsc.load_gather(n_v, [jnp.full((LANES,), batch + k, jnp.int32)])
            start = start + jnp.where(lane > k, bk + 1, 0)
        start = jnp.clip(start, 0, NUM_CHUNKS)
        npk = jnp.minimum(n, m_v[...])
        dup[pl.ds(0, LANES)] = start
        dup[pl.ds(LANES, LANES)] = start
        dup[pl.ds(2 * LANES, LANES)] = npk
        dup[pl.ds(3 * LANES, LANES)] = npk
        bidx = jnp.zeros((LANES,), jnp.int32) + LANES + tile
        s_t = plsc.load_gather(dup, [bidx])
        npk_t = plsc.load_gather(dup, [bidx + 2 * LANES])
        idx_vec = jnp.where((lane < npk_t) & (lane < MAXP), s_t + lane, ZROW)
        ha = pltpu.async_copy(ma_hbm.at[idx_vec], rows_a, sem_a)
        hb = pltpu.async_copy(mb_hbm.at[idx_vec], rows_b, sem_b)
        ha.wait()
        hb.wait()
        for r in range(MAXP):
            for g in range(NG):
                stage[pl.ds(r * EMB + g * LANES, LANES)] = (
                    rows_a[r, pl.ds(g * LANES, LANES)]
                    + rows_b[r, pl.ds(g * LANES, LANES)]
                )
        pltpu.sync_copy(
            stage, out_hbm.at[pl.ds(tile * MAXP * EMB, MAXP * EMB)]
        )


def kernel(x, chunk_size, n_peaks, max_n_peaks):
    batch, length, embed = x.shape
    xf = x.reshape(-1, embed)

    n32 = n_peaks.astype(jnp.int32)
    n_pad = (
        jnp.zeros((LANES,), jnp.int32)
        .at[:batch]
        .set(n32)
        .at[batch : 2 * batch]
        .set(n32)
    )
    maxv = jnp.full((LANES,), max_n_peaks, dtype=jnp.int32)
    inv = 1.0 / jnp.asarray(chunk_size, jnp.float32)
    scale1 = inv.reshape(1)
    scale16 = jnp.full((LANES,), inv, jnp.float32)

    mesh = plsc.VectorSubcoreMesh(core_axis_name="c", subcore_axis_name="s")
    sc_params = pltpu.CompilerParams(needs_layout_passes=False)

    means_a = pl.pallas_call(
        _tc_reduce_body,
        grid=(C0 // CPB,),
        in_specs=[
            pl.BlockSpec(memory_space=pltpu.SMEM),
            pl.BlockSpec((CPB * CHUNK, embed), lambda c: (c, 0)),
        ],
        out_specs=pl.BlockSpec((TROWS, embed), lambda c: (0, 0)),
        out_shape=jax.ShapeDtypeStruct((TROWS, embed), jnp.float32),
    )(scale1, xf)

    means_b_flat = pl.kernel(
        _sc_reduce_body,
        out_type=jax.ShapeDtypeStruct((TROWS * embed,), jnp.float32),
        mesh=mesh,
        compiler_params=sc_params,
        scratch_types=[
            pltpu.VMEM((PIECE, embed), jnp.float32),
            pltpu.VMEM((PIECE, embed), jnp.float32),
            pltpu.VMEM((PIECE, embed), jnp.float32),
            pltpu.VMEM((PIECE, embed), jnp.float32),
            pltpu.VMEM((embed,), jnp.float32),
            pltpu.VMEM((16, embed), jnp.float32),
            pltpu.VMEM((embed,), jnp.float32),
            pltpu.VMEM((C0 * embed,), jnp.float32),
            pltpu.VMEM((LANES,), jnp.float32),
            pltpu.VMEM_SHARED((H, 16, embed), jnp.float32),
            pltpu.SemaphoreType.DMA,
            pltpu.SemaphoreType.DMA,
            pltpu.SemaphoreType.DMA,
            pltpu.SemaphoreType.DMA,
        ],
    )(xf, scale16)
    means_b = means_b_flat.reshape(TROWS, embed)

    out_flat = pl.kernel(
        functools.partial(_sc_assemble_body, batch),
        out_type=jax.ShapeDtypeStruct((batch * MAXP * embed,), jnp.float32),
        mesh=mesh,
        compiler_params=sc_params,
        scratch_types=[
            pltpu.VMEM((LANES,), jnp.int32),
            pltpu.VMEM((LANES,), jnp.int32),
            pltpu.VMEM((4 * LANES,), jnp.int32),
            pltpu.VMEM((LANES, embed), jnp.float32),
            pltpu.VMEM((LANES, embed), jnp.float32),
            pltpu.VMEM((MAXP * embed,), jnp.float32),
            pltpu.SemaphoreType.DMA,
            pltpu.SemaphoreType.DMA,
        ],
    )(n_pad, maxv, means_a, means_b)
    return out_flat.reshape(batch, MAXP, embed)


# speedup vs baseline: 1.0148x; 1.0148x over previous
import functools

import jax
import jax.numpy as jnp
from jax import lax
from jax.experimental import pallas as pl
from jax.experimental.pallas import tpu as pltpu
from jax.experimental.pallas import tpu_sc as plsc

CHUNK = 2048
MAXP = 7
LANES = 16
NUM_CHUNKS = 36
H = 3
C0 = NUM_CHUNKS - 2 * H
CPB = 2
TROWS = NUM_CHUNKS + 8
ZROW = TROWS - 1
ROWS_PER_TILE = CHUNK // 16
PIECE = 32
NPIECE = ROWS_PER_TILE // PIECE
EMB = 512
NG = EMB // LANES


def _tree_sum(vals):
    while len(vals) > 1:
        pairs = [vals[i] + vals[i + 1] for i in range(0, len(vals) - 1, 2)]
        if len(vals) % 2:
            pairs.append(vals[-1])
        vals = pairs
    return vals[0]


def _acc_piece(buf, acc_ref):
    for g in range(NG):
        vals = [buf[r, pl.ds(g * LANES, LANES)] for r in range(PIECE)]
        plsc.addupdate(acc_ref.at[pl.ds(g * LANES, LANES)], _tree_sum(vals))


def _zero_fill(ref, nvec):
    for i in range(nvec):
        ref[pl.ds(i * LANES, LANES)] = jnp.zeros((LANES,), jnp.float32)


def _sc_reduce_body(
    x_hbm, scale_hbm, out_hbm,
    buf0, buf1, buf2, buf3, acc_ref, comb, rowbuf, zbuf, scale_v, partials,
    sem0, sem1, sem2, sem3,
):
    core = lax.axis_index("c")
    tile = lax.axis_index("s")
    pltpu.sync_copy(scale_hbm, scale_v)
    bufs = (buf0, buf1, buf2, buf3)
    sems = (sem0, sem1, sem2, sem3)
    base0 = (C0 + core * H) * CHUNK + tile * ROWS_PER_TILE

    def src(chunk_j, p):
        return x_hbm.at[pl.ds(base0 + chunk_j * CHUNK + p * PIECE, PIECE)]

    for p in range(3):
        pltpu.async_copy(src(0, p), bufs[p], sems[p])

    def chunk_body(j, carry):
        pltpu.async_copy(src(j, 3), bufs[3], sems[3])
        _zero_fill(acc_ref, NG)
        for p in range(4):
            pltpu.make_async_copy(src(j, p), bufs[p], sems[p]).wait()
            _acc_piece(bufs[p], acc_ref)
            if p < 3:

                @pl.when(j < H - 1)
                def _(p=p):
                    pltpu.async_copy(src(j + 1, p), bufs[p], sems[p])
        pltpu.sync_copy(acc_ref, partials.at[j, tile])
        return carry

    lax.fori_loop(0, H, chunk_body, 0)
    plsc.subcore_barrier()

    @pl.when(tile < H)
    def _():
        pltpu.sync_copy(partials.at[tile], comb)
        sv = scale_v[...]
        for g in range(NG):
            vals = [comb[r, pl.ds(g * LANES, LANES)] for r in range(16)]
            rowbuf[pl.ds(g * LANES, LANES)] = _tree_sum(vals) * sv
        row = C0 + core * H + tile
        pltpu.sync_copy(rowbuf, out_hbm.at[pl.ds(row * EMB, EMB)])

    @pl.when((core == 0) & (tile == 15))
    def _():
        _zero_fill(zbuf, C0 * EMB // LANES)
        pltpu.sync_copy(zbuf, out_hbm.at[pl.ds(0, C0 * EMB)])

    @pl.when((core == 1) & (tile == 15))
    def _():
        npad = TROWS - NUM_CHUNKS
        _zero_fill(zbuf, npad * EMB // LANES)
        pltpu.sync_copy(
            zbuf.at[pl.ds(0, npad * EMB)],
            out_hbm.at[pl.ds(NUM_CHUNKS * EMB, npad * EMB)],
        )


def _tc_reduce_body(scale_ref, x_ref, out_ref):
    c = pl.program_id(0)

    @pl.when(c == 0)
    def _():
        out_ref[pl.ds(C0, TROWS - C0), :] = jnp.zeros(
            (TROWS - C0, EMB), jnp.float32
        )

    blk = x_ref[...].reshape(CPB, CHUNK, EMB)
    sums = jnp.sum(blk, axis=1) * scale_ref[0]
    for k in range(CPB):
        out_ref[pl.ds(c * CPB + k, 1), :] = sums[k : k + 1]


def _sc_assemble_body(
    batch, n_hbm, m_hbm, ma_hbm, mb_hbm, out_hbm,
    n_v, m_v, dup, rows_a, rows_b, stage, sem_a, sem_b,
):
    core = lax.axis_index("c")
    tile = lax.axis_index("s")

    @pl.when((core == 0) & (tile < batch))
    def _():
        pltpu.sync_copy(n_hbm, n_v)
        pltpu.sync_copy(m_hbm, m_v)
        n = n_v[...]
        lane = lax.iota(jnp.int32, LANES)
        start = jnp.zeros((LANES,), jnp.int32)
        for k in range(batch):
            bk = plsc.load_gather(n_v, [jnp.full((LANES,), batch + k, jnp.int32)])
            start = start + jnp.where(lane > k, bk + 1, 0)
        start = jnp.clip(start, 0, NUM_CHUNKS)
        npk = jnp.minimum(n, m_v[...])
        dup[pl.ds(0, LANES)] = start
        dup[pl.ds(LANES, LANES)] = start
        dup[pl.ds(2 * LANES, LANES)] = npk
        dup[pl.ds(3 * LANES, LANES)] = npk
        bidx = jnp.zeros((LANES,), jnp.int32) + LANES + tile
        s_t = plsc.load_gather(dup, [bidx])
        npk_t = plsc.load_gather(dup, [bidx + 2 * LANES])
        idx_vec = jnp.where((lane < npk_t) & (lane < MAXP), s_t + lane, ZROW)
        ha = pltpu.async_copy(ma_hbm.at[idx_vec], rows_a, sem_a)
        hb = pltpu.async_copy(mb_hbm.at[idx_vec], rows_b, sem_b)
        ha.wait()
        hb.wait()
        for r in range(MAXP):
            for g in range(NG):
                stage[pl.ds(r * EMB + g * LANES, LANES)] = (
                    rows_a[r, pl.ds(g * LANES, LANES)]
                    + rows_b[r, pl.ds(g * LANES, LANES)]
                )
        pltpu.sync_copy(
            stage, out_hbm.at[pl.ds(tile * MAXP * EMB, MAXP * EMB)]
        )


def kernel(x, chunk_size, n_peaks, max_n_peaks):
    batch, length, embed = x.shape
    xf = x.reshape(-1, embed)

    n32 = n_peaks.astype(jnp.int32)
    n_pad = (
        jnp.zeros((LANES,), jnp.int32)
        .at[:batch]
        .set(n32)
        .at[batch : 2 * batch]
        .set(n32)
    )
    maxv = jnp.full((LANES,), max_n_peaks, dtype=jnp.int32)
    inv = 1.0 / jnp.asarray(chunk_size, jnp.float32)
    scale1 = inv.reshape(1)
    scale16 = jnp.full((LANES,), inv, jnp.float32)

    mesh = plsc.VectorSubcoreMesh(core_axis_name="c", subcore_axis_name="s")
    sc_params = pltpu.CompilerParams(needs_layout_passes=False)

    means_b_flat = pl.kernel(
        _sc_reduce_body,
        out_type=jax.ShapeDtypeStruct((TROWS * embed,), jnp.float32),
        mesh=mesh,
        compiler_params=sc_params,
        scratch_types=[
            pltpu.VMEM((PIECE, embed), jnp.float32),
            pltpu.VMEM((PIECE, embed), jnp.float32),
            pltpu.VMEM((PIECE, embed), jnp.float32),
            pltpu.VMEM((PIECE, embed), jnp.float32),
            pltpu.VMEM((embed,), jnp.float32),
            pltpu.VMEM((16, embed), jnp.float32),
            pltpu.VMEM((embed,), jnp.float32),
            pltpu.VMEM((C0 * embed,), jnp.float32),
            pltpu.VMEM((LANES,), jnp.float32),
            pltpu.VMEM_SHARED((H, 16, embed), jnp.float32),
            pltpu.SemaphoreType.DMA,
            pltpu.SemaphoreType.DMA,
            pltpu.SemaphoreType.DMA,
            pltpu.SemaphoreType.DMA,
        ],
    )(xf, scale16)
    means_b = means_b_flat.reshape(TROWS, embed)

    means_a = pl.pallas_call(
        _tc_reduce_body,
        grid=(C0 // CPB,),
        in_specs=[
            pl.BlockSpec(memory_space=pltpu.SMEM),
            pl.BlockSpec((CPB * CHUNK, embed), lambda c: (c, 0)),
        ],
        out_specs=pl.BlockSpec((TROWS, embed), lambda c: (0, 0)),
        out_shape=jax.ShapeDtypeStruct((TROWS, embed), jnp.float32),
    )(scale1, xf)

    out_flat = pl.kernel(
        functools.partial(_sc_assemble_body, batch),
        out_type=jax.ShapeDtypeStruct((batch * MAXP * embed,), jnp.float32),
        mesh=mesh,
        compiler_params=sc_params,
        scratch_types=[
            pltpu.VMEM((LANES,), jnp.int32),
            pltpu.VMEM((LANES,), jnp.int32),
            pltpu.VMEM((4 * LANES,), jnp.int32),
            pltpu.VMEM((LANES, embed), jnp.float32),
            pltpu.VMEM((LANES, embed), jnp.float32),
            pltpu.VMEM((MAXP * embed,), jnp.float32),
            pltpu.SemaphoreType.DMA,
            pltpu.SemaphoreType.DMA,
        ],
    )(n_pad, maxv, means_a, means_b)
    return out_flat.reshape(batch, MAXP, embed)


# speedup vs baseline: 1.2890x; 1.2702x over previous
import functools

import jax
import jax.numpy as jnp
from jax import lax
from jax.experimental import pallas as pl
from jax.experimental.pallas import tpu as pltpu
from jax.experimental.pallas import tpu_sc as plsc

CHUNK = 2048
MAXP = 7
LANES = 16


def _sc_routing_body(num_chunks, zrow, batch, n_hbm, m_hbm, out_hbm, n_v, m_v, out_v):
    @pl.when((lax.axis_index("c") == 0) & (lax.axis_index("s") == 0))
    def _():
        pltpu.sync_copy(n_hbm, n_v)
        pltpu.sync_copy(m_hbm, m_v)
        n = n_v[...]
        lane = lax.iota(jnp.int32, LANES)
        start = jnp.zeros((LANES,), jnp.int32)
        for k in range(batch):
            bk = plsc.load_gather(n_v, [jnp.full((LANES,), batch + k, jnp.int32)])
            start = start + jnp.where(lane > k, bk + 1, 0)
        start = jnp.clip(start, 0, num_chunks)
        npk = jnp.minimum(n, m_v[...])
        for j in range(MAXP):
            valid = (npk > j) & (lane < batch)
            out_v[j, :] = jnp.where(valid, start + j, zrow)
        pltpu.sync_copy(out_v, out_hbm)


def _tc_body(num_chunks, zrow, batch, cpb, src_ref, scale_ref, x_ref, out_ref, means):
    c = pl.program_id(0)

    @pl.when(c == 0)
    def _():
        means[pl.ds(num_chunks, zrow + 1 - num_chunks), :] = jnp.zeros(
            (zrow + 1 - num_chunks, means.shape[1]), jnp.float32
        )

    blk = x_ref[...].reshape(cpb, CHUNK, means.shape[1])
    sums = jnp.sum(blk, axis=1) * scale_ref[0]
    for k in range(cpb):
        means[pl.ds(c * cpb + k, 1), :] = sums[k : k + 1]

    @pl.when(c == num_chunks // cpb - 1)
    def _():
        for i in range(batch):
            for j in range(MAXP):
                v = src_ref[j * LANES + i]
                out_ref[pl.ds(i, 1), pl.ds(j, 1), :] = means[pl.ds(v, 1), :].reshape(
                    1, 1, means.shape[1]
                )


def kernel(x, chunk_size, n_peaks, max_n_peaks):
    batch, length, embed = x.shape
    xf = x.reshape(-1, embed)
    num_chunks = xf.shape[0] // CHUNK
    zrow = num_chunks + MAXP

    n32 = n_peaks.astype(jnp.int32)
    n_pad = (
        jnp.zeros((LANES,), jnp.int32)
        .at[:batch]
        .set(n32)
        .at[batch : 2 * batch]
        .set(n32)
    )
    maxv = jnp.full((LANES,), max_n_peaks, dtype=jnp.int32)

    mesh = plsc.VectorSubcoreMesh(core_axis_name="c", subcore_axis_name="s")
    src = pl.kernel(
        functools.partial(_sc_routing_body, num_chunks, zrow, batch),
        out_type=jax.ShapeDtypeStruct((MAXP, LANES), jnp.int32),
        mesh=mesh,
        compiler_params=pltpu.CompilerParams(needs_layout_passes=False),
        scratch_types=[
            pltpu.VMEM((LANES,), jnp.int32),
            pltpu.VMEM((LANES,), jnp.int32),
            pltpu.VMEM((MAXP, LANES), jnp.int32),
        ],
    )(n_pad, maxv)

    scale = (1.0 / jnp.asarray(chunk_size, jnp.float32)).reshape(1)

    cpb = 2
    out = pl.pallas_call(
        functools.partial(_tc_body, num_chunks, zrow, batch, cpb),
        grid=(num_chunks // cpb,),
        in_specs=[
            pl.BlockSpec(memory_space=pltpu.SMEM),
            pl.BlockSpec(memory_space=pltpu.SMEM),
            pl.BlockSpec((cpb * CHUNK, embed), lambda c: (c, 0)),
        ],
        out_specs=pl.BlockSpec((batch, MAXP, embed), lambda c: (0, 0, 0)),
        out_shape=jax.ShapeDtypeStruct((batch, MAXP, embed), jnp.float32),
        scratch_shapes=[pltpu.VMEM((zrow + 1, embed), jnp.float32)],
    )(src.reshape(-1), scale, xf)
    return out
